# baseline (speedup 1.0000x reference)
import numpy as np
import jax
import jax.numpy as jnp
from jax import lax
from jax.experimental import pallas as pl
from jax.experimental.pallas import tpu as pltpu

_C = 256
_NL = 4
_GROUPS = 32
_GS = _C // _GROUPS
_EPS = 1e-5
_HO = 80

_GEOM = []
for _H, _B in ((64, 72), (32, 40), (16, 24), (8, 16), (4, 8)):
    _NP = (_H + 2) * _H
    _GEOM.append((_H, _B, _NP, _NP + 2 * _B))


def _np_shift_masks(H, NP):
    j = np.arange(NP)
    mm = (j % H != 0).astype(np.float32)[:, None]
    mp = (j % H != H - 1).astype(np.float32)[:, None]
    return mm, mp


_SHIFT_MASKS_NP = [_np_shift_masks(H, NP) for (H, B, NP, NT) in _GEOM]
_AGG_NP = (np.arange(_C)[:, None] // _GS
           == np.arange(_C)[None, :] // _GS).astype(np.float32)


def _conv9(cur, mm, mp, w_ref, row0, B, NP, W, NCO):
    acc = None
    t = 0
    for dh in (-1, 0, 1):
        for dw in (-1, 0, 1):
            r0 = B + dh * W + dw
            xs = lax.slice(cur, (r0, 0), (r0 + NP, _C))
            if dw == -1:
                xs = xs * mm
            elif dw == 1:
                xs = xs * mp
            wt = w_ref[0, row0 + t * _C: row0 + (t + 1) * _C, 0:NCO]
            p = jnp.dot(xs, wt, preferred_element_type=jnp.float32)
            acc = p if acc is None else acc + p
            t += 1
    return acc


def _layer(cur, mm, mp, geom, z, aggm, w_ref, i, cp):
    H, B, NP, NT = geom
    HW = H * H
    n = float(_GS * HW)
    b, bgm, bg2 = cp[0:1, :], cp[1:2, :], cp[2:3, :]
    o = _conv9(cur, mm, mp, w_ref, (i * 9) * _C, B, NP, H, _C)
    o = lax.slice(o, (H, 0), (H + HW, _C))
    csum = jnp.sum(o, axis=0, keepdims=True)
    csq = jnp.sum(o * o, axis=0, keepdims=True)
    r = jnp.dot(jnp.concatenate([csum, csq, csum * b], axis=0), aggm,
                preferred_element_type=jnp.float32)
    mu = r[0:1, :] / n + bgm
    e2 = r[1:2, :] / n + 2.0 * r[2:3, :] / n + bg2
    s = lax.rsqrt(e2 - mu * mu + _EPS) * cp[3:4, :]
    sh = (b - mu) * s + cp[4:5, :]
    return jnp.concatenate([z, jnp.maximum(o * s + sh, 0.0), z], axis=0)


def _body(*refs):
    xs = list(refs[0:5])
    mms = [refs[5 + 2 * l] for l in range(5)]
    mps = [refs[6 + 2 * l] for l in range(5)]
    agg_ref, tw, taff, hw_ref, hb_ref, sc_ref = refs[15:21]
    outs = list(refs[21:26])
    aggm = agg_ref[:, :]
    pid = pl.program_id(0)
    for l in range(5):
        geom = _GEOM[l]
        H, B, NP, NT = geom
        HW = H * H
        mm = mms[l][:, :]
        mp = mps[l][:, :]
        z = jnp.zeros((B + H, _C), jnp.float32)
        cur = jnp.concatenate([z, xs[l][:, :].T, z], axis=0)
        for i in range(_NL):
            cur = _layer(cur, mm, mp, geom, z, aggm, tw, i,
                         taff[0, 8 * i: 8 * i + 5, :])
        y = _conv9(cur, mm, mp, hw_ref, 0, B, NP, H, _HO)
        y = lax.slice(y, (H, 0), (H + HW, _HO)) + hb_ref[0, 0:1, :]
        sval = sc_ref[l:l + 1, :]
        colid = lax.broadcasted_iota(jnp.int32, y.shape, 1)
        pick = jnp.logical_and(colid < 4, pid == 1)
        y = jnp.where(pick, jnp.maximum(y * sval, 0.0), y)
        outs[l][0, :, :] = y


def _to_matmul_w(w):
    if w.ndim == 5:
        nl = w.shape[0]
        return w.transpose(0, 3, 4, 2, 1).reshape(nl * 9 * _C, w.shape[1])
    return w.transpose(2, 3, 1, 0).reshape(9 * _C, w.shape[0])


def _affine_bundle(b, g, bt):
    bgm = jnp.mean(b.reshape(_NL, _GROUPS, _GS), axis=2, keepdims=True)
    bgm = jnp.broadcast_to(bgm, (_NL, _GROUPS, _GS)).reshape(_NL, _C)
    bg2 = jnp.mean((b * b).reshape(_NL, _GROUPS, _GS), axis=2, keepdims=True)
    bg2 = jnp.broadcast_to(bg2, (_NL, _GROUPS, _GS)).reshape(_NL, _C)
    zero = jnp.zeros_like(b)
    rows = jnp.stack([b, bgm, bg2, g, bt, zero, zero, zero], axis=1)
    return rows.reshape(_NL * 8, _C)


def _pad_cols(w, n):
    return jnp.pad(w, ((0, 0), (0, n - w.shape[1])))


def kernel(p3, p4, p5, p6, p7, cls_w, cls_b, cls_gn_g, cls_gn_b,
           box_w, box_b, box_gn_g, box_gn_b,
           score_w, score_b, pred_w, pred_b, ctr_w, ctr_b, scales):
    feats = [p3, p4, p5, p6, p7]
    args = [f.reshape(_C, _GEOM[l][0] ** 2) for l, f in enumerate(feats)]
    full = [pl.BlockSpec(a.shape, lambda i: (0, 0)) for a in args]
    for mm, mp in _SHIFT_MASKS_NP:
        args += [jnp.asarray(mm), jnp.asarray(mp)]
        full += [pl.BlockSpec(mm.shape, lambda i: (0, 0)),
                 pl.BlockSpec(mp.shape, lambda i: (0, 0))]
    args.append(jnp.asarray(_AGG_NP))
    full.append(pl.BlockSpec((_C, _C), lambda i: (0, 0)))

    bp_w = jnp.concatenate([_to_matmul_w(pred_w), _to_matmul_w(ctr_w)],
                           axis=1)
    args += [
        jnp.stack([_to_matmul_w(cls_w), _to_matmul_w(box_w)]),
        jnp.stack([_affine_bundle(cls_b, cls_gn_g, cls_gn_b),
                   _affine_bundle(box_b, box_gn_g, box_gn_b)]),
        jnp.stack([_to_matmul_w(score_w), _pad_cols(bp_w, _HO)]),
        jnp.stack([score_b[None, :],
                   _pad_cols(jnp.concatenate([pred_b, ctr_b])[None, :], _HO)]),
        scales[:, None],
    ]
    full += [
        pl.BlockSpec((1, _NL * 9 * _C, _C), lambda i: (i, 0, 0)),
        pl.BlockSpec((1, _NL * 8, _C), lambda i: (i, 0, 0)),
        pl.BlockSpec((1, 9 * _C, _HO), lambda i: (i, 0, 0)),
        pl.BlockSpec((1, 1, _HO), lambda i: (i, 0, 0)),
        pl.BlockSpec((5, 1), lambda i: (0, 0)),
    ]
    out_shape = [jax.ShapeDtypeStruct((2, H * H, _HO), jnp.float32)
                 for (H, B, NP, NT) in _GEOM]
    out_specs = [pl.BlockSpec((1, H * H, _HO), lambda i: (i, 0, 0))
                 for (H, B, NP, NT) in _GEOM]
    res = pl.pallas_call(
        _body,
        grid=(2,),
        in_specs=full,
        out_specs=out_specs,
        out_shape=out_shape,
        compiler_params=pltpu.CompilerParams(
            dimension_semantics=("parallel",),
            vmem_limit_bytes=100 * 1024 * 1024),
    )(*args)

    def _img(y, cols):
        H = int(round(y.shape[0] ** 0.5))
        return y.reshape(H, H, -1).transpose(2, 0, 1)[None, :cols]

    logits = [_img(res[l][0], 80) for l in range(5)]
    bbox = [_img(res[l][1, :, 0:4], 4) for l in range(5)]
    ctr = [_img(res[l][1, :, 4:5], 1) for l in range(5)]
    return tuple(logits + bbox + ctr)


# speedup vs baseline: 1.2498x; 1.2498x over previous
import numpy as np
import jax
import jax.numpy as jnp
from jax import lax
from jax.experimental import pallas as pl
from jax.experimental.pallas import tpu as pltpu

_C = 256
_NL = 4
_GROUPS = 32
_GS = _C // _GROUPS
_EPS = 1e-5
_PREC = lax.Precision.DEFAULT

_GEOM = []
for _H, _B in ((64, 72), (32, 40), (16, 24), (8, 16), (4, 8)):
    _NP = (_H + 2) * _H
    _GEOM.append((_H, _B, _NP, _NP + 2 * _B))


def _np_shift_masks(H, NP):
    j = np.arange(NP)
    mm = (j % H != 0).astype(np.float32)[:, None]
    mp = (j % H != H - 1).astype(np.float32)[:, None]
    return mm, mp


_SHIFT_MASKS_NP = [_np_shift_masks(H, NP) for (H, B, NP, NT) in _GEOM]
_AGG_NP = (np.arange(_C)[:, None] // _GS
           == np.arange(_C)[None, :] // _GS).astype(np.float32)


def _conv9(cur, mm, mp, w_ref, row0, B, NP, W):
    acc = None
    t = 0
    for dh in (-1, 0, 1):
        for dw in (-1, 0, 1):
            r0 = B + dh * W + dw
            xs = lax.slice(cur, (r0, 0), (r0 + NP, _C))
            if dw == -1:
                xs = xs * mm
            elif dw == 1:
                xs = xs * mp
            wt = w_ref[row0 + t * _C: row0 + (t + 1) * _C, :]
            p = jnp.dot(xs, wt, preferred_element_type=jnp.float32,
                        precision=_PREC)
            acc = p if acc is None else acc + p
            t += 1
    return acc


def _layer(cur, mm, mp, geom, z, aggm, w_ref, i, cp):
    H, B, NP, NT = geom
    HW = H * H
    n = float(_GS * HW)
    b, bgm, bg2 = cp[0:1, :], cp[1:2, :], cp[2:3, :]
    o = _conv9(cur, mm, mp, w_ref, (i * 9) * _C, B, NP, H)
    o = lax.slice(o, (H, 0), (H + HW, _C))
    csum = jnp.sum(o, axis=0, keepdims=True)
    csq = jnp.sum(o * o, axis=0, keepdims=True)
    r = jnp.dot(jnp.concatenate([csum, csq, csum * b], axis=0), aggm,
                preferred_element_type=jnp.float32, precision=_PREC)
    mu = r[0:1, :] / n + bgm
    e2 = r[1:2, :] / n + 2.0 * r[2:3, :] / n + bg2
    s = lax.rsqrt(e2 - mu * mu + _EPS) * cp[3:4, :]
    sh = (b - mu) * s + cp[4:5, :]
    return jnp.concatenate([z, jnp.maximum(o * s + sh, 0.0), z], axis=0)


def _make_body(is_box):
    def body(*refs):
        xs = list(refs[0:5])
        mms = [refs[5 + 2 * l] for l in range(5)]
        mps = [refs[6 + 2 * l] for l in range(5)]
        agg_ref, tw, tb, hw_ref, hb_ref = refs[15:20]
        if is_box:
            sc_ref = refs[20]
            outs = list(refs[21:26])
        else:
            sc_ref = None
            outs = list(refs[20:25])
        aggm = agg_ref[:, :]
        for l in range(5):
            geom = _GEOM[l]
            H, B, NP, NT = geom
            HW = H * H
            mm = mms[l][:, :]
            mp = mps[l][:, :]
            z = jnp.zeros((B + H, _C), jnp.float32)
            cur = jnp.concatenate([z, xs[l][:, :].T, z], axis=0)
            for i in range(_NL):
                cur = _layer(cur, mm, mp, geom, z, aggm, tw, i,
                             tb[8 * i: 8 * i + 5, :])
            y = _conv9(cur, mm, mp, hw_ref, 0, B, NP, H)
            y = lax.slice(y, (H, 0), (H + HW, y.shape[1])) + hb_ref[0:1, :]
            if is_box:
                sval = sc_ref[l:l + 1, :]
                colid = lax.broadcasted_iota(jnp.int32, y.shape, 1)
                y = jnp.where(colid < 4, jnp.maximum(y * sval, 0.0), y)
            outs[l][:, :] = y
    return body


def _to_matmul_w(w):
    if w.ndim == 5:
        nl = w.shape[0]
        return w.transpose(0, 3, 4, 2, 1).reshape(nl * 9 * _C, w.shape[1])
    return w.transpose(2, 3, 1, 0).reshape(9 * _C, w.shape[0])


def _affine_bundle(b, g, bt):
    bgm = jnp.mean(b.reshape(_NL, _GROUPS, _GS), axis=2, keepdims=True)
    bgm = jnp.broadcast_to(bgm, (_NL, _GROUPS, _GS)).reshape(_NL, _C)
    bg2 = jnp.mean((b * b).reshape(_NL, _GROUPS, _GS), axis=2, keepdims=True)
    bg2 = jnp.broadcast_to(bg2, (_NL, _GROUPS, _GS)).reshape(_NL, _C)
    zero = jnp.zeros_like(b)
    rows = jnp.stack([b, bgm, bg2, g, bt, zero, zero, zero], axis=1)
    return rows.reshape(_NL * 8, _C)


def kernel(p3, p4, p5, p6, p7, cls_w, cls_b, cls_gn_g, cls_gn_b,
           box_w, box_b, box_gn_g, box_gn_b,
           score_w, score_b, pred_w, pred_b, ctr_w, ctr_b, scales):
    feats = [p3, p4, p5, p6, p7]
    base = [f.reshape(_C, _GEOM[l][0] ** 2) for l, f in enumerate(feats)]
    for mm, mp in _SHIFT_MASKS_NP:
        base += [jnp.asarray(mm), jnp.asarray(mp)]
    base.append(jnp.asarray(_AGG_NP))

    def _call(is_box, tower_args, head_width):
        out_shape = [jax.ShapeDtypeStruct((H * H, head_width), jnp.float32)
                     for (H, B, NP, NT) in _GEOM]
        return pl.pallas_call(
            _make_body(is_box),
            out_shape=out_shape,
            compiler_params=pltpu.CompilerParams(
                vmem_limit_bytes=100 * 1024 * 1024),
        )(*(base + tower_args))

    logits_i = _call(False, [
        _to_matmul_w(cls_w), _affine_bundle(cls_b, cls_gn_g, cls_gn_b),
        _to_matmul_w(score_w), score_b[None, :]], 80)
    bc_i = _call(True, [
        _to_matmul_w(box_w), _affine_bundle(box_b, box_gn_g, box_gn_b),
        jnp.concatenate([_to_matmul_w(pred_w), _to_matmul_w(ctr_w)], axis=1),
        jnp.concatenate([pred_b, ctr_b])[None, :],
        scales[:, None]], 5)

    def _img(y, l, cols):
        H = _GEOM[l][0]
        return y.reshape(H, H, -1).transpose(2, 0, 1)[None, :cols]

    logits = [_img(logits_i[l], l, 80) for l in range(5)]
    bbox = [_img(bc_i[l][:, 0:4], l, 4) for l in range(5)]
    ctr = [_img(bc_i[l][:, 4:5], l, 1) for l in range(5)]
    return tuple(logits + bbox + ctr)
